# Initial kernel scaffold; baseline (speedup 1.0000x reference)
import jax
import jax.numpy as jnp
from jax.experimental import pallas as pl


def kernel(atom_types, pos, table):
    raise NotImplementedError("write your pallas kernel here")


# speedup vs baseline: 1.5007x; 1.5007x over previous
import functools

import jax
import jax.numpy as jnp
from jax import lax
from jax.experimental import pallas as pl
from jax.experimental.pallas import tpu as pltpu
from jax.experimental.pallas import tpu_sc as plsc

_C = 128


@functools.lru_cache(maxsize=None)
def _build_sc_gather(n, d, dtype_name):
    dtype = jnp.dtype(dtype_name)
    info = plsc.get_sparse_core_info()
    nc, ns = info.num_cores, info.num_subcores
    nw = nc * ns
    full = n // _C
    tail = n % _C
    base_chunks = full // nw
    extra = full % nw
    tail_w = full % nw

    mesh = plsc.VectorSubcoreMesh(core_axis_name="c", subcore_axis_name="s")

    scratch = [
        pltpu.VMEM((_C,), jnp.int32),
        pltpu.VMEM((_C, d), dtype),
        pltpu.SemaphoreType.DMA,
    ]
    if tail:
        scratch += [
            pltpu.VMEM((tail,), jnp.int32),
            pltpu.VMEM((tail, d), dtype),
        ]

    @functools.partial(
        pl.kernel,
        mesh=mesh,
        out_type=jax.ShapeDtypeStruct((n, d), dtype),
        scratch_types=scratch,
    )
    def gather_kernel(idx_hbm, table_hbm, out_hbm, idx_v, rows_v, sem,
                      *tail_scratch):
        w = lax.axis_index("s") * nc + lax.axis_index("c")
        nchunks = base_chunks + jnp.where(w < extra, 1, 0)

        def body(i, carry):
            base = (w + i * nw) * _C
            pltpu.sync_copy(idx_hbm.at[pl.ds(base, _C)], idx_v)
            pltpu.async_copy(table_hbm.at[idx_v], rows_v, sem).wait()
            pltpu.sync_copy(rows_v, out_hbm.at[pl.ds(base, _C), :])
            return carry

        lax.fori_loop(0, nchunks, body, 0)

        if tail:
            idxt_v, rowst_v = tail_scratch

            @pl.when(w == tail_w)
            def _():
                base = full * _C
                pltpu.sync_copy(idx_hbm.at[pl.ds(base, tail)], idxt_v)
                pltpu.async_copy(table_hbm.at[idxt_v], rowst_v, sem).wait()
                pltpu.sync_copy(rowst_v, out_hbm.at[pl.ds(base, tail), :])

    return gather_kernel


def kernel(atom_types, pos, table):
    idx = jnp.reshape(atom_types, (-1,))
    tab = table.astype(pos.dtype)
    n = idx.shape[0]
    d = tab.shape[1]
    out = _build_sc_gather(n, d, str(tab.dtype))(idx, tab)
    return (out, out)


# speedup vs baseline: 1.5095x; 1.0058x over previous
import functools

import jax
import jax.numpy as jnp
from jax import lax
from jax.experimental import pallas as pl
from jax.experimental.pallas import tpu as pltpu
from jax.experimental.pallas import tpu_sc as plsc

_C = 128
_NBUF = 4
_DIST = 2


@functools.lru_cache(maxsize=None)
def _build_sc_gather(n, d, dtype_name):
    dtype = jnp.dtype(dtype_name)
    info = plsc.get_sparse_core_info()
    nc, ns = info.num_cores, info.num_subcores
    nw = nc * ns
    full = n // _C
    tail = n % _C
    base = full // nw
    extra = full % nw
    assert base >= _NBUF and tail % 8 == 0 and extra < nw - 1

    len_lo = base * _C
    len_hi = (base + 1) * _C
    len_last = base * _C + tail

    mesh = plsc.VectorSubcoreMesh(core_axis_name="c", subcore_axis_name="s")

    scratch = [
        pltpu.VMEM((len_hi,), jnp.int32),
        pltpu.VMEM((_C, d), dtype),
        pltpu.VMEM((_C, d), dtype),
        pltpu.VMEM((_C, d), dtype),
        pltpu.VMEM((_C, d), dtype),
        pltpu.VMEM((max(tail, 8), d), dtype),
        pltpu.SemaphoreType.DMA((_NBUF,)),
        pltpu.SemaphoreType.DMA((_NBUF,)),
    ]

    @functools.partial(
        pl.kernel,
        mesh=mesh,
        out_type=jax.ShapeDtypeStruct((n, d), dtype),
        scratch_types=scratch,
    )
    def gather_kernel(idx_hbm, table_hbm, out_hbm, idx_all,
                      r0, r1, r2, r3, rows_t, gsem, ssem):
        rows = (r0, r1, r2, r3)
        w = lax.axis_index("s") * nc + lax.axis_index("c")
        s = base * w + jnp.minimum(w, extra)
        idx_start = s * _C

        @pl.when(w < extra)
        def _():
            pltpu.sync_copy(idx_hbm.at[pl.ds(idx_start, len_hi)],
                            idx_all.at[pl.ds(0, len_hi)])

        @pl.when(jnp.logical_and(w >= extra, w < nw - 1))
        def _():
            pltpu.sync_copy(idx_hbm.at[pl.ds(idx_start, len_lo)],
                            idx_all.at[pl.ds(0, len_lo)])

        @pl.when(w == nw - 1)
        def _():
            pltpu.sync_copy(idx_hbm.at[pl.ds(idx_start, len_last)],
                            idx_all.at[pl.ds(0, len_last)])

        def gather_async(c, b):
            return pltpu.async_copy(
                table_hbm.at[idx_all.at[pl.ds(c * _C, _C)]], rows[b],
                gsem.at[b])

        def wait_gather(c, b):
            pltpu.make_async_copy(
                table_hbm.at[idx_all.at[pl.ds(c * _C, _C)]], rows[b],
                gsem.at[b]).wait()

        def wait_store(b):
            pltpu.make_async_copy(rows[b], out_hbm.at[pl.ds(0, _C), :],
                                  ssem.at[b]).wait()

        for c in range(_DIST):
            gather_async(c, c % _NBUF)

        for c in range(base):
            b = c % _NBUF
            wait_gather(c, b)
            pltpu.async_copy(rows[b], out_hbm.at[pl.ds((s + c) * _C, _C), :],
                             ssem.at[b])
            c2 = c + _DIST
            b2 = c2 % _NBUF
            if c2 < base:
                if c2 >= _NBUF:
                    wait_store(b2)
                gather_async(c2, b2)
            elif c2 == base:
                @pl.when(w < extra)
                def _(c2=c2, b2=b2):
                    wait_store(b2)
                    gather_async(c2, b2)

        @pl.when(w < extra)
        def _():
            b = base % _NBUF
            wait_gather(base, b)
            pltpu.sync_copy(rows[b], out_hbm.at[pl.ds((s + base) * _C, _C), :])
            for bb in range(_NBUF):
                if bb != base % _NBUF:
                    wait_store(bb)

        @pl.when(w >= extra)
        def _():
            for bb in range(_NBUF):
                wait_store(bb)

        if tail:
            @pl.when(w == nw - 1)
            def _():
                pltpu.async_copy(
                    table_hbm.at[idx_all.at[pl.ds(base * _C, tail)]],
                    rows_t.at[pl.ds(0, tail), :], gsem.at[0]).wait()
                pltpu.sync_copy(rows_t.at[pl.ds(0, tail), :],
                                out_hbm.at[pl.ds(full * _C, tail), :])

    return gather_kernel


def kernel(atom_types, pos, table):
    idx = jnp.reshape(atom_types, (-1,))
    tab = table.astype(pos.dtype)
    n = idx.shape[0]
    d = tab.shape[1]
    out = _build_sc_gather(n, d, str(tab.dtype))(idx, tab)
    return (out, out)


# speedup vs baseline: 3.5784x; 2.3706x over previous
import functools

import jax
import jax.numpy as jnp
from jax import lax
from jax.experimental import pallas as pl
from jax.experimental.pallas import tpu as pltpu
from jax.experimental.pallas import tpu_sc as plsc

_C = 128
_NBUF = 4
_DIST = 2


@functools.lru_cache(maxsize=None)
def _build_sc_gather(n, v, d, dtype_name):
    dtype = jnp.dtype(dtype_name)
    info = plsc.get_sparse_core_info()
    nc, ns = info.num_cores, info.num_subcores
    nw = nc * ns
    full = n // _C
    tail = n % _C
    base = full // nw
    extra = full % nw
    assert base >= _NBUF and tail % 8 == 0 and extra < nw - 1

    len_lo = base * _C
    len_hi = (base + 1) * _C
    len_last = base * _C + tail

    mesh = plsc.VectorSubcoreMesh(core_axis_name="c", subcore_axis_name="s")

    scratch = [
        pltpu.VMEM((len_hi,), jnp.int32),
        pltpu.VMEM((_C, d), dtype),
        pltpu.VMEM((_C, d), dtype),
        pltpu.VMEM((_C, d), dtype),
        pltpu.VMEM((_C, d), dtype),
        pltpu.VMEM((max(tail, 8), d), dtype),
        pltpu.SemaphoreType.DMA((_NBUF,)),
        pltpu.SemaphoreType.DMA((_NBUF,)),
        pltpu.VMEM_SHARED((v, d), dtype),
    ]

    @functools.partial(
        pl.kernel,
        mesh=mesh,
        out_type=jax.ShapeDtypeStruct((n, d), dtype),
        scratch_types=scratch,
    )
    def gather_kernel(idx_hbm, table_hbm, out_hbm, idx_all,
                      r0, r1, r2, r3, rows_t, gsem, ssem, tab_sp):
        rows = (r0, r1, r2, r3)
        sid = lax.axis_index("s")
        w = sid * nc + lax.axis_index("c")
        s = base * w + jnp.minimum(w, extra)
        idx_start = s * _C

        @pl.when(sid == 0)
        def _():
            pltpu.sync_copy(table_hbm, tab_sp)
        plsc.subcore_barrier()

        @pl.when(w < extra)
        def _():
            pltpu.sync_copy(idx_hbm.at[pl.ds(idx_start, len_hi)],
                            idx_all.at[pl.ds(0, len_hi)])

        @pl.when(jnp.logical_and(w >= extra, w < nw - 1))
        def _():
            pltpu.sync_copy(idx_hbm.at[pl.ds(idx_start, len_lo)],
                            idx_all.at[pl.ds(0, len_lo)])

        @pl.when(w == nw - 1)
        def _():
            pltpu.sync_copy(idx_hbm.at[pl.ds(idx_start, len_last)],
                            idx_all.at[pl.ds(0, len_last)])

        def gather_async(c, b):
            return pltpu.async_copy(
                tab_sp.at[idx_all.at[pl.ds(c * _C, _C)]], rows[b],
                gsem.at[b])

        def wait_gather(c, b):
            pltpu.make_async_copy(
                tab_sp.at[idx_all.at[pl.ds(c * _C, _C)]], rows[b],
                gsem.at[b]).wait()

        def wait_store(b):
            pltpu.make_async_copy(rows[b], out_hbm.at[pl.ds(0, _C), :],
                                  ssem.at[b]).wait()

        for c in range(_DIST):
            gather_async(c, c % _NBUF)

        for c in range(base):
            b = c % _NBUF
            wait_gather(c, b)
            pltpu.async_copy(rows[b], out_hbm.at[pl.ds((s + c) * _C, _C), :],
                             ssem.at[b])
            c2 = c + _DIST
            b2 = c2 % _NBUF
            if c2 < base:
                if c2 >= _NBUF:
                    wait_store(b2)
                gather_async(c2, b2)
            elif c2 == base:
                @pl.when(w < extra)
                def _(c2=c2, b2=b2):
                    wait_store(b2)
                    gather_async(c2, b2)

        @pl.when(w < extra)
        def _():
            b = base % _NBUF
            wait_gather(base, b)
            pltpu.sync_copy(rows[b], out_hbm.at[pl.ds((s + base) * _C, _C), :])
            for bb in range(_NBUF):
                if bb != base % _NBUF:
                    wait_store(bb)

        @pl.when(w >= extra)
        def _():
            for bb in range(_NBUF):
                wait_store(bb)

        if tail:
            @pl.when(w == nw - 1)
            def _():
                pltpu.async_copy(
                    tab_sp.at[idx_all.at[pl.ds(base * _C, tail)]],
                    rows_t.at[pl.ds(0, tail), :], gsem.at[0]).wait()
                pltpu.sync_copy(rows_t.at[pl.ds(0, tail), :],
                                out_hbm.at[pl.ds(full * _C, tail), :])

    return gather_kernel


def kernel(atom_types, pos, table):
    idx = jnp.reshape(atom_types, (-1,))
    tab = table.astype(pos.dtype)
    n = idx.shape[0]
    v, d = tab.shape
    out = _build_sc_gather(n, v, d, str(tab.dtype))(idx, tab)
    return (out, out)
